# baseline (speedup 1.0000x reference)
import functools

import jax
import jax.numpy as jnp
from jax import lax
from jax.experimental import pallas as pl
from jax.experimental.pallas import tpu as pltpu
from jax.experimental.pallas import tpu_sc as plsc

B, P, C, O = 32, 16320, 21, 50
OP = 64
NCH = 8
BLK = P // NCH
THRESHOLD = 0.5
NEGPOS_RATIO = 3
VAR0, VAR1 = 0.1, 0.2


def _truth_rows(t):
    return (t[0:1, :], t[1:2, :], t[2:3, :], t[3:4, :], t[4:5, :],
            t[5:6, :] > 0.5)


def _match_a(t8_ref, pr_ref, bto_ref, bti_ref, bpi_ref, bpo_s, bpi_s):
    c = pl.program_id(1)
    tx1, ty1, tx2, ty2, _, valid = _truth_rows(t8_ref[0])
    area_a = (tx2 - tx1) * (ty2 - ty1)
    pr = pr_ref[...]
    pw = pr[:, 2:3]
    ph = pr[:, 3:4]
    px1 = pr[:, 0:1] - pw * 0.5
    px2 = pr[:, 0:1] + pw * 0.5
    py1 = pr[:, 1:2] - ph * 0.5
    py2 = pr[:, 1:2] + ph * 0.5
    iw = jnp.maximum(jnp.minimum(px2, tx2) - jnp.maximum(px1, tx1), 0.0)
    ih = jnp.maximum(jnp.minimum(py2, ty2) - jnp.maximum(py1, ty1), 0.0)
    inter = iw * ih
    ov = inter / (area_a + pw * ph - inter)
    ov = jnp.where(valid, ov, -1.0)
    bto_ref[0] = jnp.max(ov, axis=1, keepdims=True)
    bti_ref[0] = jnp.argmax(ov, axis=1, keepdims=True).astype(jnp.int32)

    @pl.when(c == 0)
    def _():
        bpo_s[...] = jnp.full((1, OP), -2.0, jnp.float32)
        bpi_s[...] = jnp.zeros((1, OP), jnp.int32)

    mx = jnp.max(ov, axis=0, keepdims=True)
    amx = jnp.argmax(ov, axis=0, keepdims=True).astype(jnp.int32) + c * BLK
    upd = mx > bpo_s[...]
    bpi_new = jnp.where(upd, amx, bpi_s[...])
    bpo_s[...] = jnp.where(upd, mx, bpo_s[...])
    bpi_s[...] = bpi_new
    bpi_ref[0] = bpi_new


def _match_b(t8_ref, pr_ref, loc_ref, conf_ref, bto_ref, bti_ref, bpi_ref,
             lc_ref, stats_ref):
    c = pl.program_id(1)
    tx1, ty1, tx2, ty2, lab, valid = _truth_rows(t8_ref[0])
    bpi = bpi_ref[0]
    pidx = jax.lax.broadcasted_iota(jnp.int32, (BLK, OP), 0) + c * BLK
    oid = jax.lax.broadcasted_iota(jnp.int32, (BLK, OP), 1)
    hit = (bpi == pidx) & valid
    last_j = jnp.max(jnp.where(hit, oid, -1), axis=1, keepdims=True)
    forced = last_j >= 0
    bto = jnp.where(forced, 2.0, bto_ref[0])
    bti = jnp.where(forced, last_j, bti_ref[0])
    oh = (oid == bti).astype(jnp.float32)
    gx1 = jnp.sum(oh * tx1, axis=1, keepdims=True)
    gy1 = jnp.sum(oh * ty1, axis=1, keepdims=True)
    gx2 = jnp.sum(oh * tx2, axis=1, keepdims=True)
    gy2 = jnp.sum(oh * ty2, axis=1, keepdims=True)
    glab = jnp.sum(oh * lab, axis=1, keepdims=True)
    conf_t = jnp.where(bto < THRESHOLD, 0, glab.astype(jnp.int32))
    pos = conf_t > 0

    pr = pr_ref[...]
    pw = pr[:, 2:3]
    ph = pr[:, 3:4]
    ecx = ((gx1 + gx2) * 0.5 - pr[:, 0:1]) / (VAR0 * pw)
    ecy = ((gy1 + gy2) * 0.5 - pr[:, 1:2]) / (VAR0 * ph)
    ew = jnp.log(jnp.maximum((gx2 - gx1) / pw, 1e-30)) / VAR1
    eh = jnp.log(jnp.maximum((gy2 - gy1) / ph, 1e-30)) / VAR1
    loc = loc_ref[0]

    def huber(d):
        ad = jnp.abs(d)
        return jnp.where(ad < 1.0, 0.5 * d * d, ad - 0.5)

    sl1 = (huber(loc[:, 0:1] - ecx) + huber(loc[:, 1:2] - ecy)
           + huber(loc[:, 2:3] - ew) + huber(loc[:, 3:4] - eh))
    loss_l = jnp.sum(jnp.where(pos, sl1, 0.0))
    npos = jnp.sum(pos.astype(jnp.float32))

    cf = conf_ref[0]
    m = jnp.max(cf, axis=1, keepdims=True)
    lse = jnp.log(jnp.sum(jnp.exp(cf - m), axis=1, keepdims=True)) + m
    cid = jax.lax.broadcasted_iota(jnp.int32, (BLK, C), 1)
    gat = jnp.sum(jnp.where(cid == conf_t, cf, 0.0), axis=1, keepdims=True)
    ce = lse - gat
    pos_ce = jnp.sum(jnp.where(pos, ce, 0.0))
    lc_ref[0] = jnp.where(pos, 0.0, ce)

    part = jnp.concatenate([
        loss_l.reshape(1, 1), pos_ce.reshape(1, 1), npos.reshape(1, 1),
        jnp.zeros((1, 5), jnp.float32)], axis=1)

    @pl.when(c == 0)
    def _():
        stats_ref[0] = jnp.zeros((1, 8), jnp.float32)

    stats_ref[0] = stats_ref[0] + part


NC, NS, L = 2, 16, 16
NV = P // L


def _sc_topk(lc_hbm, kb_hbm, out_hbm, row_v, kb_v, out_v):
    wid = lax.axis_index("s") * NC + lax.axis_index("c")
    pltpu.sync_copy(lc_hbm.at[wid], row_v)
    pltpu.sync_copy(kb_hbm.at[wid], kb_v)
    kvec = kb_v[...]

    def count_gt(midv):
        tfv = plsc.bitcast(midv, jnp.float32)

        def body(j, acc):
            v = row_v[pl.ds(j * L, L)]
            return acc + plsc.all_reduce_population_count(v > tfv)

        return lax.fori_loop(0, NV, body, jnp.zeros((L,), jnp.int32))

    def bs_body(_, carry):
        lo, hi = carry
        mid = lo + jnp.right_shift(hi - lo, 1)
        take_hi = count_gt(mid) < kvec
        return (jnp.where(take_hi, lo, mid + 1),
                jnp.where(take_hi, mid, hi))

    zi = jnp.zeros((L,), jnp.int32)
    lo, _ = lax.fori_loop(0, 31, bs_body, (zi, zi + 0x7F800000))
    tfv = plsc.bitcast(lo, jnp.float32)

    def fbody(j, carry):
        s, c, tm = carry
        v = row_v[pl.ds(j * L, L)]
        gt = v > tfv
        return (s + jnp.where(gt, v, 0.0),
                c + gt.astype(jnp.float32),
                jnp.maximum(tm, jnp.where(v == tfv, v, 0.0)))

    zf = jnp.zeros((L,), jnp.float32)
    s, c, tm = lax.fori_loop(0, NV, fbody, (zf, zf, zf))
    out_v[pl.ds(0, L)] = s
    out_v[pl.ds(L, L)] = c
    out_v[pl.ds(2 * L, L)] = tm
    pltpu.sync_copy(out_v, out_hbm.at[wid])


def _combine_d(stats_ref, tk_ref, out_ref):
    tk = tk_ref[...]
    sum_gt = jnp.sum(tk[:, 0:L], axis=1, keepdims=True)
    cnt_gt = jnp.sum(tk[:, L:2 * L], axis=1, keepdims=True)
    tval = jnp.max(tk[:, 2 * L:3 * L], axis=1, keepdims=True)
    npos = stats_ref[:, 2:3]
    kf = jnp.minimum(jnp.float32(NEGPOS_RATIO) * npos, jnp.float32(P - 1))
    topk = sum_gt + (kf - cnt_gt) * tval
    n = jnp.sum(npos)
    loss_l = jnp.sum(stats_ref[:, 0:1])
    loss_c = jnp.sum(stats_ref[:, 1:2] + topk)
    out_ref[...] = jnp.concatenate([
        (loss_l / n).reshape(1), (loss_c / n).reshape(1),
        jnp.zeros((6,), jnp.float32)])


def _topk_c(lc_ref, stats_ref, out_ref):
    v = lc_ref[...]
    vbits = jax.lax.bitcast_convert_type(v, jnp.int32)
    npos = stats_ref[:, 2:3]
    k = jnp.minimum(jnp.float32(NEGPOS_RATIO) * npos, jnp.float32(P - 1))
    k = k.astype(jnp.int32)

    def body(_, carry):
        lo, hi = carry
        mid = lo + jax.lax.div(hi - lo, 2)
        cnt = jnp.sum((vbits > mid).astype(jnp.int32), axis=1, keepdims=True)
        take_hi = cnt < k
        return (jnp.where(take_hi, lo, mid + 1),
                jnp.where(take_hi, mid, hi))

    lo0 = jnp.zeros((B, 1), jnp.int32)
    hi0 = jnp.full((B, 1), 0x7F800000, jnp.int32)
    lo, _ = jax.lax.fori_loop(0, 31, body, (lo0, hi0))
    t = jax.lax.bitcast_convert_type(lo, jnp.float32)
    gt = vbits > lo
    sum_gt = jnp.sum(jnp.where(gt, v, 0.0), axis=1, keepdims=True)
    cnt_gt = jnp.sum(gt.astype(jnp.float32), axis=1, keepdims=True)
    topk = sum_gt + (k.astype(jnp.float32) - cnt_gt) * t

    n = jnp.sum(npos)
    loss_l = jnp.sum(stats_ref[:, 0:1])
    loss_c = jnp.sum(stats_ref[:, 1:2] + topk)
    out_ref[...] = jnp.concatenate([
        (loss_l / n).reshape(1), (loss_c / n).reshape(1),
        jnp.zeros((6,), jnp.float32)])


@jax.jit
def kernel(loc_data, conf_data, priors, targets):
    boxes = targets[:, :, :4]
    labels = targets[:, :, 4]
    t8 = jnp.zeros((B, 8, OP), jnp.float32)
    t8 = t8.at[:, 0:4, :O].set(jnp.transpose(boxes, (0, 2, 1)))
    t8 = t8.at[:, 4, :O].set(labels)
    t8 = t8.at[:, 5, :O].set(1.0)

    bto, bti, bpi = pl.pallas_call(
        _match_a,
        grid=(B, NCH),
        in_specs=[
            pl.BlockSpec((1, 8, OP), lambda b, c: (b, 0, 0)),
            pl.BlockSpec((BLK, 4), lambda b, c: (c, 0)),
        ],
        out_specs=[
            pl.BlockSpec((1, BLK, 1), lambda b, c: (b, c, 0)),
            pl.BlockSpec((1, BLK, 1), lambda b, c: (b, c, 0)),
            pl.BlockSpec((1, 1, OP), lambda b, c: (b, 0, 0)),
        ],
        out_shape=[
            jax.ShapeDtypeStruct((B, P, 1), jnp.float32),
            jax.ShapeDtypeStruct((B, P, 1), jnp.int32),
            jax.ShapeDtypeStruct((B, 1, OP), jnp.int32),
        ],
        scratch_shapes=[
            pltpu.VMEM((1, OP), jnp.float32),
            pltpu.VMEM((1, OP), jnp.int32),
        ],
    )(t8, priors)

    lc, stats = pl.pallas_call(
        _match_b,
        grid=(B, NCH),
        in_specs=[
            pl.BlockSpec((1, 8, OP), lambda b, c: (b, 0, 0)),
            pl.BlockSpec((BLK, 4), lambda b, c: (c, 0)),
            pl.BlockSpec((1, BLK, 4), lambda b, c: (b, c, 0)),
            pl.BlockSpec((1, BLK, C), lambda b, c: (b, c, 0)),
            pl.BlockSpec((1, BLK, 1), lambda b, c: (b, c, 0)),
            pl.BlockSpec((1, BLK, 1), lambda b, c: (b, c, 0)),
            pl.BlockSpec((1, 1, OP), lambda b, c: (b, 0, 0)),
        ],
        out_specs=[
            pl.BlockSpec((1, BLK, 1), lambda b, c: (b, c, 0)),
            pl.BlockSpec((1, 1, 8), lambda b, c: (b, 0, 0)),
        ],
        out_shape=[
            jax.ShapeDtypeStruct((B, P, 1), jnp.float32),
            jax.ShapeDtypeStruct((B, 1, 8), jnp.float32),
        ],
    )(t8, priors, loc_data, conf_data, bto, bti, bpi)

    stats2 = stats.reshape(B, 8)
    kb = jnp.minimum(jnp.float32(NEGPOS_RATIO) * stats2[:, 2:3],
                     jnp.float32(P - 1)).astype(jnp.int32)
    kb = jnp.broadcast_to(kb, (B, L))

    sc_topk = functools.partial(
        pl.kernel,
        mesh=plsc.VectorSubcoreMesh(core_axis_name="c", subcore_axis_name="s"),
        compiler_params=pltpu.CompilerParams(needs_layout_passes=False),
        out_type=jax.ShapeDtypeStruct((B, 3 * L), jnp.float32),
        scratch_types=[
            pltpu.VMEM((P,), jnp.float32),
            pltpu.VMEM((L,), jnp.int32),
            pltpu.VMEM((3 * L,), jnp.float32),
        ],
    )(_sc_topk)
    tk = sc_topk(lc.reshape(B, P), kb)

    out = pl.pallas_call(
        _combine_d,
        in_specs=[
            pl.BlockSpec((B, 8), lambda: (0, 0)),
            pl.BlockSpec((B, 3 * L), lambda: (0, 0)),
        ],
        out_specs=pl.BlockSpec((8,), lambda: (0,)),
        out_shape=jax.ShapeDtypeStruct((8,), jnp.float32),
    )(stats2, tk)
    return out[0:2]


# speedup vs baseline: 3.8797x; 3.8797x over previous
import functools

import jax
import jax.numpy as jnp
from jax import lax
from jax.experimental import pallas as pl
from jax.experimental.pallas import tpu as pltpu
from jax.experimental.pallas import tpu_sc as plsc

B, P, C, O = 32, 16320, 21, 50
OP = 64
PP = 16384
NCH = 8
LBLK = PP // NCH
CP = 24
THRESHOLD = 0.5
NEGPOS_RATIO = 3
VAR0, VAR1 = 0.1, 0.2


def _match_a(t8b_ref, pra_ref, bto_ref, bti_ref, bpi_ref, bpo_s, bpi_s):
    c = pl.program_id(1)
    tb = t8b_ref[0]
    tx1 = tb[:, 0:1]
    ty1 = tb[:, 1:2]
    tx2 = tb[:, 2:3]
    ty2 = tb[:, 3:4]
    valid = tb[:, 5:6] > 0.5
    area_a = (tx2 - tx1) * (ty2 - ty1)
    pra = pra_ref[...]
    px1 = pra[0:1, :]
    py1 = pra[1:2, :]
    px2 = pra[2:3, :]
    py2 = pra[3:4, :]
    areab = pra[4:5, :]
    iw = jnp.maximum(jnp.minimum(px2, tx2) - jnp.maximum(px1, tx1), 0.0)
    ih = jnp.maximum(jnp.minimum(py2, ty2) - jnp.maximum(py1, ty1), 0.0)
    inter = iw * ih
    ov = inter / (area_a + areab - inter)
    ov = jnp.where(valid, ov, -1.0)
    bto_ref[0] = jnp.max(ov, axis=0, keepdims=True)
    bti_ref[0] = jnp.argmax(ov, axis=0, keepdims=True).astype(jnp.int32)

    @pl.when(c == 0)
    def _():
        bpo_s[...] = jnp.full((OP, 1), -2.0, jnp.float32)
        bpi_s[...] = jnp.zeros((OP, 1), jnp.int32)

    mx = jnp.max(ov, axis=1, keepdims=True)
    amx = jnp.argmax(ov, axis=1, keepdims=True).astype(jnp.int32) + c * LBLK
    upd = mx > bpo_s[...]
    bpi_new = jnp.where(upd, amx, bpi_s[...])
    bpo_s[...] = jnp.where(upd, mx, bpo_s[...])
    bpi_s[...] = bpi_new
    bpi_ref[0] = bpi_new


def _match_b(t8_ref, t8b_ref, prb_ref, loc_ref, conf_ref, bto_ref, bti_ref,
             bpi_ref, lc_ref, stats_ref):
    c = pl.program_id(1)
    tmat = t8_ref[0]
    valid = t8b_ref[0][:, 5:6] > 0.5
    bpi = bpi_ref[0]
    pidx = jax.lax.broadcasted_iota(jnp.int32, (OP, LBLK), 1) + c * LBLK
    oid = jax.lax.broadcasted_iota(jnp.int32, (OP, LBLK), 0)
    hit = (bpi == pidx) & valid
    last_j = jnp.max(jnp.where(hit, oid, -1), axis=0, keepdims=True)
    forced = last_j >= 0
    bto = jnp.where(forced, 2.0, bto_ref[0])
    bti = jnp.where(forced, last_j, bti_ref[0])
    oh = (oid == bti).astype(jnp.float32)
    g = jax.lax.dot_general(tmat, oh, (((1,), (0,)), ((), ())),
                            preferred_element_type=jnp.float32)
    gx1 = g[0:1, :]
    gy1 = g[1:2, :]
    gx2 = g[2:3, :]
    gy2 = g[3:4, :]
    glab = g[4:5, :]
    conf_t = jnp.where(bto < THRESHOLD, 0, glab.astype(jnp.int32))
    pos = conf_t > 0

    prb = prb_ref[...]
    ecx = ((gx1 + gx2) * 0.5 - prb[0:1, :]) * prb[2:3, :]
    ecy = ((gy1 + gy2) * 0.5 - prb[1:2, :]) * prb[3:4, :]
    ew = jnp.log(jnp.maximum((gx2 - gx1) * prb[4:5, :], 1e-30)) * (1.0 / VAR1)
    eh = jnp.log(jnp.maximum((gy2 - gy1) * prb[5:6, :], 1.0e-30)) * (1.0 / VAR1)
    locr = loc_ref[0]

    def huber(d):
        ad = jnp.abs(d)
        return jnp.where(ad < 1.0, 0.5 * d * d, ad - 0.5)

    sl1 = (huber(locr[0:1, :] - ecx) + huber(locr[1:2, :] - ecy)
           + huber(locr[2:3, :] - ew) + huber(locr[3:4, :] - eh))
    loss_l = jnp.sum(jnp.where(pos, sl1, 0.0))
    npos = jnp.sum(pos.astype(jnp.float32))

    cf = conf_ref[0]
    lse = jnp.log(jnp.sum(jnp.exp(cf), axis=0, keepdims=True))
    cid = jax.lax.broadcasted_iota(jnp.int32, (CP, LBLK), 0)
    gat = jnp.sum(jnp.where(cid == conf_t, cf, 0.0), axis=0, keepdims=True)
    ce = lse - gat
    pos_ce = jnp.sum(jnp.where(pos, ce, 0.0))
    padmask = pidx[0:1, :] >= P
    lc_ref[0] = jnp.where(pos | padmask, 0.0, ce)

    part = jnp.concatenate([
        loss_l.reshape(1, 1), pos_ce.reshape(1, 1), npos.reshape(1, 1),
        jnp.zeros((1, 5), jnp.float32)], axis=1)

    @pl.when(c == 0)
    def _():
        stats_ref[0] = jnp.zeros((1, 8), jnp.float32)

    stats_ref[0] = stats_ref[0] + part


NC, NS, L = 2, 16, 16
NV = PP // L


def _sc_topk(lc_hbm, kb_hbm, out_hbm, row_v, kb_v, out_v):
    wid = lax.axis_index("s") * NC + lax.axis_index("c")
    pltpu.sync_copy(lc_hbm.at[wid], row_v)
    pltpu.sync_copy(kb_hbm.at[wid], kb_v)
    kvec = kb_v[...]

    def count_gt(midv):
        tfv = plsc.bitcast(midv, jnp.float32)

        def body(j, acc):
            v = row_v[pl.ds(j * L, L)]
            return acc + plsc.all_reduce_population_count(v > tfv)

        return lax.fori_loop(0, NV, body, jnp.zeros((L,), jnp.int32))

    def bs_body(_, carry):
        lo, hi = carry
        mid = lo + jnp.right_shift(hi - lo, 1)
        take_hi = count_gt(mid) < kvec
        return (jnp.where(take_hi, lo, mid + 1),
                jnp.where(take_hi, mid, hi))

    zi = jnp.zeros((L,), jnp.int32)
    lo, _ = lax.fori_loop(0, 31, bs_body, (zi, zi + 0x7F800000))
    tfv = plsc.bitcast(lo, jnp.float32)

    def fbody(j, carry):
        s, cn, tm = carry
        v = row_v[pl.ds(j * L, L)]
        gt = v > tfv
        return (s + jnp.where(gt, v, 0.0),
                cn + gt.astype(jnp.float32),
                jnp.maximum(tm, jnp.where(v == tfv, v, 0.0)))

    zf = jnp.zeros((L,), jnp.float32)
    s, cn, tm = lax.fori_loop(0, NV, fbody, (zf, zf, zf))
    out_v[pl.ds(0, L)] = s
    out_v[pl.ds(L, L)] = cn
    out_v[pl.ds(2 * L, L)] = tm
    pltpu.sync_copy(out_v, out_hbm.at[wid])


def _combine_d(stats_ref, tk_ref, out_ref):
    tk = tk_ref[...]
    sum_gt = jnp.sum(tk[:, 0:L], axis=1, keepdims=True)
    cnt_gt = jnp.sum(tk[:, L:2 * L], axis=1, keepdims=True)
    tval = jnp.max(tk[:, 2 * L:3 * L], axis=1, keepdims=True)
    npos = stats_ref[:, 2:3]
    kf = jnp.minimum(jnp.float32(NEGPOS_RATIO) * npos, jnp.float32(P - 1))
    topk = sum_gt + (kf - cnt_gt) * tval
    n = jnp.sum(npos)
    loss_l = jnp.sum(stats_ref[:, 0:1])
    loss_c = jnp.sum(stats_ref[:, 1:2] + topk)
    out_ref[...] = jnp.concatenate([
        (loss_l / n).reshape(1), (loss_c / n).reshape(1),
        jnp.zeros((6,), jnp.float32)])


@jax.jit
def kernel(loc_data, conf_data, priors, targets):
    f32 = jnp.float32
    boxes = targets[:, :, :4]
    labels = targets[:, :, 4]
    t8 = jnp.zeros((B, 8, OP), f32)
    t8 = t8.at[:, 0:4, :O].set(jnp.transpose(boxes, (0, 2, 1)))
    t8 = t8.at[:, 4, :O].set(labels)
    t8 = t8.at[:, 5, :O].set(1.0)
    t8b = jnp.transpose(t8, (0, 2, 1))

    cx, cy, w, h = priors[:, 0], priors[:, 1], priors[:, 2], priors[:, 3]
    pra = jnp.zeros((8, PP), f32)
    pra = pra.at[0, :P].set(cx - w * 0.5)
    pra = pra.at[1, :P].set(cy - h * 0.5)
    pra = pra.at[2, :P].set(cx + w * 0.5)
    pra = pra.at[3, :P].set(cy + h * 0.5)
    pra = pra.at[4, :P].set(w * h)
    pra = pra.at[0:4, P:].set(-10.0)
    prb = jnp.ones((8, PP), f32)
    prb = prb.at[0, :P].set(cx)
    prb = prb.at[1, :P].set(cy)
    prb = prb.at[2, :P].set(1.0 / (VAR0 * w))
    prb = prb.at[3, :P].set(1.0 / (VAR0 * h))
    prb = prb.at[4, :P].set(1.0 / w)
    prb = prb.at[5, :P].set(1.0 / h)

    loc_t = jnp.zeros((B, 8, PP), f32)
    loc_t = loc_t.at[:, 0:4, :P].set(jnp.transpose(loc_data, (0, 2, 1)))
    conf_t_in = jnp.full((B, CP, PP), -1e30, f32)
    conf_t_in = conf_t_in.at[:, :C, :P].set(jnp.transpose(conf_data, (0, 2, 1)))

    bto, bti, bpi = pl.pallas_call(
        _match_a,
        grid=(B, NCH),
        in_specs=[
            pl.BlockSpec((1, OP, 8), lambda b, c: (b, 0, 0)),
            pl.BlockSpec((8, LBLK), lambda b, c: (0, c)),
        ],
        out_specs=[
            pl.BlockSpec((1, 1, LBLK), lambda b, c: (b * NCH + c, 0, 0)),
            pl.BlockSpec((1, 1, LBLK), lambda b, c: (b * NCH + c, 0, 0)),
            pl.BlockSpec((1, OP, 1), lambda b, c: (b, 0, 0)),
        ],
        out_shape=[
            jax.ShapeDtypeStruct((B * NCH, 1, LBLK), f32),
            jax.ShapeDtypeStruct((B * NCH, 1, LBLK), jnp.int32),
            jax.ShapeDtypeStruct((B, OP, 1), jnp.int32),
        ],
        scratch_shapes=[
            pltpu.VMEM((OP, 1), f32),
            pltpu.VMEM((OP, 1), jnp.int32),
        ],
    )(t8b, pra)

    lc, stats = pl.pallas_call(
        _match_b,
        grid=(B, NCH),
        in_specs=[
            pl.BlockSpec((1, 8, OP), lambda b, c: (b, 0, 0)),
            pl.BlockSpec((1, OP, 8), lambda b, c: (b, 0, 0)),
            pl.BlockSpec((8, LBLK), lambda b, c: (0, c)),
            pl.BlockSpec((1, 8, LBLK), lambda b, c: (b, 0, c)),
            pl.BlockSpec((1, CP, LBLK), lambda b, c: (b, 0, c)),
            pl.BlockSpec((1, 1, LBLK), lambda b, c: (b * NCH + c, 0, 0)),
            pl.BlockSpec((1, 1, LBLK), lambda b, c: (b * NCH + c, 0, 0)),
            pl.BlockSpec((1, OP, 1), lambda b, c: (b, 0, 0)),
        ],
        out_specs=[
            pl.BlockSpec((1, 1, LBLK), lambda b, c: (b * NCH + c, 0, 0)),
            pl.BlockSpec((1, 1, 8), lambda b, c: (b, 0, 0)),
        ],
        out_shape=[
            jax.ShapeDtypeStruct((B * NCH, 1, LBLK), f32),
            jax.ShapeDtypeStruct((B, 1, 8), f32),
        ],
    )(t8, t8b, prb, loc_t, conf_t_in, bto, bti, bpi)

    stats2 = stats.reshape(B, 8)
    kb = jnp.minimum(jnp.float32(NEGPOS_RATIO) * stats2[:, 2:3],
                     jnp.float32(P - 1)).astype(jnp.int32)
    kb = jnp.broadcast_to(kb, (B, L))

    sc_topk = functools.partial(
        pl.kernel,
        mesh=plsc.VectorSubcoreMesh(core_axis_name="c", subcore_axis_name="s"),
        compiler_params=pltpu.CompilerParams(needs_layout_passes=False),
        out_type=jax.ShapeDtypeStruct((B, 3 * L), f32),
        scratch_types=[
            pltpu.VMEM((PP,), f32),
            pltpu.VMEM((L,), jnp.int32),
            pltpu.VMEM((3 * L,), f32),
        ],
    )(_sc_topk)
    tk = sc_topk(lc.reshape(B, PP), kb)

    out = pl.pallas_call(
        _combine_d,
        in_specs=[
            pl.BlockSpec((B, 8), lambda: (0, 0)),
            pl.BlockSpec((B, 3 * L), lambda: (0, 0)),
        ],
        out_specs=pl.BlockSpec((8,), lambda: (0,)),
        out_shape=jax.ShapeDtypeStruct((8,), f32),
    )(stats2, tk)
    return out[0:2]


# speedup vs baseline: 4.7489x; 1.2241x over previous
import functools

import jax
import jax.numpy as jnp
from jax import lax
from jax.experimental import pallas as pl
from jax.experimental.pallas import tpu as pltpu
from jax.experimental.pallas import tpu_sc as plsc

B, P, C, O = 32, 16320, 21, 50
OP = 64
PP = 16384
NCH = 8
LBLK = PP // NCH
CP = 24
THRESHOLD = 0.5
NEGPOS_RATIO = 3
VAR0, VAR1 = 0.1, 0.2


def _match_a(t8b_ref, pra_ref, bto_ref, bti_ref, bpi_ref, bpo_s, bpi_s):
    c = pl.program_id(1)
    tb = t8b_ref[0]
    tx1 = tb[:, 0:1]
    ty1 = tb[:, 1:2]
    tx2 = tb[:, 2:3]
    ty2 = tb[:, 3:4]
    valid = tb[:, 5:6] > 0.5
    area_a = (tx2 - tx1) * (ty2 - ty1)
    pra = pra_ref[...]
    px1 = pra[0:1, :]
    py1 = pra[1:2, :]
    px2 = pra[2:3, :]
    py2 = pra[3:4, :]
    areab = pra[4:5, :]
    iw = jnp.maximum(jnp.minimum(px2, tx2) - jnp.maximum(px1, tx1), 0.0)
    ih = jnp.maximum(jnp.minimum(py2, ty2) - jnp.maximum(py1, ty1), 0.0)
    inter = iw * ih
    ov = inter / (area_a + areab - inter)
    ov = jnp.where(valid, ov, -1.0)
    bto_ref[0] = jnp.max(ov, axis=0, keepdims=True)
    bti_ref[0] = jnp.argmax(ov, axis=0, keepdims=True).astype(jnp.int32)

    @pl.when(c == 0)
    def _():
        bpo_s[...] = jnp.full((OP, 1), -2.0, jnp.float32)
        bpi_s[...] = jnp.zeros((OP, 1), jnp.int32)

    mx = jnp.max(ov, axis=1, keepdims=True)
    amx = jnp.argmax(ov, axis=1, keepdims=True).astype(jnp.int32) + c * LBLK
    upd = mx > bpo_s[...]
    bpi_new = jnp.where(upd, amx, bpi_s[...])
    bpo_s[...] = jnp.where(upd, mx, bpo_s[...])
    bpi_s[...] = bpi_new
    bpi_ref[0] = bpi_new


def _match_b(t8_ref, t8b_ref, prb_ref, loc_ref, conf_ref, bto_ref, bti_ref,
             bpi_ref, lc_ref, stats_ref):
    c = pl.program_id(1)
    tmat = t8_ref[0]
    valid = t8b_ref[0][:, 5:6] > 0.5
    bpi = bpi_ref[0]
    pidx = jax.lax.broadcasted_iota(jnp.int32, (OP, LBLK), 1) + c * LBLK
    oid = jax.lax.broadcasted_iota(jnp.int32, (OP, LBLK), 0)
    hit = (bpi == pidx) & valid
    last_j = jnp.max(jnp.where(hit, oid, -1), axis=0, keepdims=True)
    forced = last_j >= 0
    bto = jnp.where(forced, 2.0, bto_ref[0])
    bti = jnp.where(forced, last_j, bti_ref[0])
    oh = (oid == bti).astype(jnp.float32)
    g = jax.lax.dot_general(tmat, oh, (((1,), (0,)), ((), ())),
                            preferred_element_type=jnp.float32)
    gx1 = g[0:1, :]
    gy1 = g[1:2, :]
    gx2 = g[2:3, :]
    gy2 = g[3:4, :]
    glab = g[4:5, :]
    conf_t = jnp.where(bto < THRESHOLD, 0, glab.astype(jnp.int32))
    pos = conf_t > 0

    prb = prb_ref[...]
    ecx = ((gx1 + gx2) * 0.5 - prb[0:1, :]) * prb[2:3, :]
    ecy = ((gy1 + gy2) * 0.5 - prb[1:2, :]) * prb[3:4, :]
    ew = jnp.log(jnp.maximum((gx2 - gx1) * prb[4:5, :], 1e-30)) * (1.0 / VAR1)
    eh = jnp.log(jnp.maximum((gy2 - gy1) * prb[5:6, :], 1.0e-30)) * (1.0 / VAR1)
    locr = loc_ref[0]

    def huber(d):
        ad = jnp.abs(d)
        return jnp.where(ad < 1.0, 0.5 * d * d, ad - 0.5)

    sl1 = (huber(locr[0:1, :] - ecx) + huber(locr[1:2, :] - ecy)
           + huber(locr[2:3, :] - ew) + huber(locr[3:4, :] - eh))
    loss_l = jnp.sum(jnp.where(pos, sl1, 0.0))
    npos = jnp.sum(pos.astype(jnp.float32))

    cf = conf_ref[0]
    lse = jnp.log(jnp.sum(jnp.exp(cf), axis=0, keepdims=True))
    cid = jax.lax.broadcasted_iota(jnp.int32, (CP, LBLK), 0)
    gat = jnp.sum(jnp.where(cid == conf_t, cf, 0.0), axis=0, keepdims=True)
    ce = lse - gat
    pos_ce = jnp.sum(jnp.where(pos, ce, 0.0))
    padmask = pidx[0:1, :] >= P
    lc_ref[0] = jnp.where(pos | padmask, 0.0, ce)

    part = jnp.concatenate([
        loss_l.reshape(1, 1), pos_ce.reshape(1, 1), npos.reshape(1, 1),
        jnp.zeros((1, 5), jnp.float32)], axis=1)

    @pl.when(c == 0)
    def _():
        stats_ref[0] = jnp.zeros((1, 8), jnp.float32)

    stats_ref[0] = stats_ref[0] + part


NC, NS, L = 2, 16, 16
NV = PP // L


def _sc_topk(lc_hbm, kb_hbm, out_hbm, row_v, kb_v, out_v):
    wid = lax.axis_index("s") * NC + lax.axis_index("c")
    pltpu.sync_copy(lc_hbm.at[wid], row_v)
    pltpu.sync_copy(kb_hbm.at[wid], kb_v)
    kvec = kb_v[...]

    def count_gt(midv):
        tfv = plsc.bitcast(midv, jnp.float32)

        def body(i, acc):
            v = row_v[pl.ds(i, L)]
            return acc + plsc.all_reduce_population_count(v > tfv)

        return plsc.parallel_loop(
            0, PP, L, unroll=8, carry=jnp.zeros((L,), jnp.int32))(body)

    def bs_body(_, carry):
        lo, hi = carry
        mid = lo + jnp.right_shift(hi - lo, 1)
        take_hi = count_gt(mid) < kvec
        return (jnp.where(take_hi, lo, mid + 1),
                jnp.where(take_hi, mid, hi))

    zi = jnp.zeros((L,), jnp.int32)
    lo, _ = lax.fori_loop(0, 31, bs_body, (zi, zi + 0x7F800000))
    tfv = plsc.bitcast(lo, jnp.float32)

    zf = jnp.zeros((L,), jnp.float32)

    def fbody(i, carry):
        s, cn, tm = carry
        v = row_v[pl.ds(i, L)]
        gt = v > tfv
        return (s + jnp.where(gt, v, 0.0),
                cn + gt.astype(jnp.float32),
                jnp.maximum(tm, jnp.where(v == tfv, v, 0.0)))

    s, cn, tm = plsc.parallel_loop(
        0, PP, L, unroll=8, carry=(zf, zf, zf))(fbody)
    out_v[pl.ds(0, L)] = s
    out_v[pl.ds(L, L)] = cn
    out_v[pl.ds(2 * L, L)] = tm
    pltpu.sync_copy(out_v, out_hbm.at[wid])


def _combine_d(stats_ref, tk_ref, out_ref):
    tk = tk_ref[...]
    sum_gt = jnp.sum(tk[:, 0:L], axis=1, keepdims=True)
    cnt_gt = jnp.sum(tk[:, L:2 * L], axis=1, keepdims=True)
    tval = jnp.max(tk[:, 2 * L:3 * L], axis=1, keepdims=True)
    npos = stats_ref[:, 2:3]
    kf = jnp.minimum(jnp.float32(NEGPOS_RATIO) * npos, jnp.float32(P - 1))
    topk = sum_gt + (kf - cnt_gt) * tval
    n = jnp.sum(npos)
    loss_l = jnp.sum(stats_ref[:, 0:1])
    loss_c = jnp.sum(stats_ref[:, 1:2] + topk)
    out_ref[...] = jnp.concatenate([
        (loss_l / n).reshape(1), (loss_c / n).reshape(1),
        jnp.zeros((6,), jnp.float32)])


@jax.jit
def kernel(loc_data, conf_data, priors, targets):
    f32 = jnp.float32
    boxes = targets[:, :, :4]
    labels = targets[:, :, 4]
    t8 = jnp.zeros((B, 8, OP), f32)
    t8 = t8.at[:, 0:4, :O].set(jnp.transpose(boxes, (0, 2, 1)))
    t8 = t8.at[:, 4, :O].set(labels)
    t8 = t8.at[:, 5, :O].set(1.0)
    t8b = jnp.transpose(t8, (0, 2, 1))

    cx, cy, w, h = priors[:, 0], priors[:, 1], priors[:, 2], priors[:, 3]
    pra = jnp.zeros((8, PP), f32)
    pra = pra.at[0, :P].set(cx - w * 0.5)
    pra = pra.at[1, :P].set(cy - h * 0.5)
    pra = pra.at[2, :P].set(cx + w * 0.5)
    pra = pra.at[3, :P].set(cy + h * 0.5)
    pra = pra.at[4, :P].set(w * h)
    pra = pra.at[0:4, P:].set(-10.0)
    prb = jnp.ones((8, PP), f32)
    prb = prb.at[0, :P].set(cx)
    prb = prb.at[1, :P].set(cy)
    prb = prb.at[2, :P].set(1.0 / (VAR0 * w))
    prb = prb.at[3, :P].set(1.0 / (VAR0 * h))
    prb = prb.at[4, :P].set(1.0 / w)
    prb = prb.at[5, :P].set(1.0 / h)

    loc_t = jnp.zeros((B, 8, PP), f32)
    loc_t = loc_t.at[:, 0:4, :P].set(jnp.transpose(loc_data, (0, 2, 1)))
    conf_t_in = jnp.full((B, CP, PP), -1e30, f32)
    conf_t_in = conf_t_in.at[:, :C, :P].set(jnp.transpose(conf_data, (0, 2, 1)))

    bto, bti, bpi = pl.pallas_call(
        _match_a,
        grid=(B, NCH),
        in_specs=[
            pl.BlockSpec((1, OP, 8), lambda b, c: (b, 0, 0)),
            pl.BlockSpec((8, LBLK), lambda b, c: (0, c)),
        ],
        out_specs=[
            pl.BlockSpec((1, 1, LBLK), lambda b, c: (b * NCH + c, 0, 0)),
            pl.BlockSpec((1, 1, LBLK), lambda b, c: (b * NCH + c, 0, 0)),
            pl.BlockSpec((1, OP, 1), lambda b, c: (b, 0, 0)),
        ],
        out_shape=[
            jax.ShapeDtypeStruct((B * NCH, 1, LBLK), f32),
            jax.ShapeDtypeStruct((B * NCH, 1, LBLK), jnp.int32),
            jax.ShapeDtypeStruct((B, OP, 1), jnp.int32),
        ],
        scratch_shapes=[
            pltpu.VMEM((OP, 1), f32),
            pltpu.VMEM((OP, 1), jnp.int32),
        ],
    )(t8b, pra)

    lc, stats = pl.pallas_call(
        _match_b,
        grid=(B, NCH),
        in_specs=[
            pl.BlockSpec((1, 8, OP), lambda b, c: (b, 0, 0)),
            pl.BlockSpec((1, OP, 8), lambda b, c: (b, 0, 0)),
            pl.BlockSpec((8, LBLK), lambda b, c: (0, c)),
            pl.BlockSpec((1, 8, LBLK), lambda b, c: (b, 0, c)),
            pl.BlockSpec((1, CP, LBLK), lambda b, c: (b, 0, c)),
            pl.BlockSpec((1, 1, LBLK), lambda b, c: (b * NCH + c, 0, 0)),
            pl.BlockSpec((1, 1, LBLK), lambda b, c: (b * NCH + c, 0, 0)),
            pl.BlockSpec((1, OP, 1), lambda b, c: (b, 0, 0)),
        ],
        out_specs=[
            pl.BlockSpec((1, 1, LBLK), lambda b, c: (b * NCH + c, 0, 0)),
            pl.BlockSpec((1, 1, 8), lambda b, c: (b, 0, 0)),
        ],
        out_shape=[
            jax.ShapeDtypeStruct((B * NCH, 1, LBLK), f32),
            jax.ShapeDtypeStruct((B, 1, 8), f32),
        ],
    )(t8, t8b, prb, loc_t, conf_t_in, bto, bti, bpi)

    stats2 = stats.reshape(B, 8)
    kb = jnp.minimum(jnp.float32(NEGPOS_RATIO) * stats2[:, 2:3],
                     jnp.float32(P - 1)).astype(jnp.int32)
    kb = jnp.broadcast_to(kb, (B, L))

    sc_topk = functools.partial(
        pl.kernel,
        mesh=plsc.VectorSubcoreMesh(core_axis_name="c", subcore_axis_name="s"),
        compiler_params=pltpu.CompilerParams(needs_layout_passes=False),
        out_type=jax.ShapeDtypeStruct((B, 3 * L), f32),
        scratch_types=[
            pltpu.VMEM((PP,), f32),
            pltpu.VMEM((L,), jnp.int32),
            pltpu.VMEM((3 * L,), f32),
        ],
    )(_sc_topk)
    tk = sc_topk(lc.reshape(B, PP), kb)

    out = pl.pallas_call(
        _combine_d,
        in_specs=[
            pl.BlockSpec((B, 8), lambda: (0, 0)),
            pl.BlockSpec((B, 3 * L), lambda: (0, 0)),
        ],
        out_specs=pl.BlockSpec((8,), lambda: (0,)),
        out_shape=jax.ShapeDtypeStruct((8,), f32),
    )(stats2, tk)
    return out[0:2]


# speedup vs baseline: 6.1068x; 1.2859x over previous
import functools

import jax
import jax.numpy as jnp
from jax import lax
from jax.experimental import pallas as pl
from jax.experimental.pallas import tpu as pltpu
from jax.experimental.pallas import tpu_sc as plsc

B, P, C, O = 32, 16320, 21, 50
OP = 64
PP = 16384
NCH = 4
LBLK = PP // NCH
CP = 24
THRESHOLD = 0.5
NEGPOS_RATIO = 3
VAR0, VAR1 = 0.1, 0.2


def _match_a(t8b_ref, pra_ref, bto_ref, bti_ref, bpi_ref, bpo_s, bpi_s):
    c = pl.program_id(1)
    tb = t8b_ref[0]
    tx1 = tb[:, 0:1]
    ty1 = tb[:, 1:2]
    tx2 = tb[:, 2:3]
    ty2 = tb[:, 3:4]
    valid = tb[:, 5:6] > 0.5
    area_a = (tx2 - tx1) * (ty2 - ty1)
    pra = pra_ref[...]
    px1 = pra[0:1, :]
    py1 = pra[1:2, :]
    px2 = pra[2:3, :]
    py2 = pra[3:4, :]
    areab = pra[4:5, :]
    iw = jnp.maximum(jnp.minimum(px2, tx2) - jnp.maximum(px1, tx1), 0.0)
    ih = jnp.maximum(jnp.minimum(py2, ty2) - jnp.maximum(py1, ty1), 0.0)
    inter = iw * ih
    ov = inter / (area_a + areab - inter)
    ov = jnp.where(valid, ov, -1.0)
    bto_ref[0] = jnp.max(ov, axis=0, keepdims=True)
    bti_ref[0] = jnp.argmax(ov, axis=0, keepdims=True).astype(jnp.int32)

    @pl.when(c == 0)
    def _():
        bpo_s[...] = jnp.full((OP, 1), -2.0, jnp.float32)
        bpi_s[...] = jnp.zeros((OP, 1), jnp.int32)

    mx = jnp.max(ov, axis=1, keepdims=True)
    amx = jnp.argmax(ov, axis=1, keepdims=True).astype(jnp.int32) + c * LBLK
    upd = mx > bpo_s[...]
    bpi_new = jnp.where(upd, amx, bpi_s[...])
    bpo_s[...] = jnp.where(upd, mx, bpo_s[...])
    bpi_s[...] = bpi_new
    bpi_ref[0] = bpi_new


def _match_b(t8_ref, t8b_ref, prb_ref, loc_ref, conf_ref, bto_ref, bti_ref,
             bpi_ref, lc_ref, stats_ref):
    c = pl.program_id(1)
    tmat = t8_ref[0]
    valid = t8b_ref[0][:, 5:6] > 0.5
    bpi = bpi_ref[0]
    pidx = jax.lax.broadcasted_iota(jnp.int32, (OP, LBLK), 1) + c * LBLK
    oid = jax.lax.broadcasted_iota(jnp.int32, (OP, LBLK), 0)
    hit = (bpi == pidx) & valid
    last_j = jnp.max(jnp.where(hit, oid, -1), axis=0, keepdims=True)
    forced = last_j >= 0
    bto = jnp.where(forced, 2.0, bto_ref[0])
    bti = jnp.where(forced, last_j, bti_ref[0])
    oh = (oid == bti).astype(jnp.float32)
    g = jax.lax.dot_general(tmat, oh, (((1,), (0,)), ((), ())),
                            preferred_element_type=jnp.float32)
    gx1 = g[0:1, :]
    gy1 = g[1:2, :]
    gx2 = g[2:3, :]
    gy2 = g[3:4, :]
    glab = g[4:5, :]
    conf_t = jnp.where(bto < THRESHOLD, 0, glab.astype(jnp.int32))
    pos = conf_t > 0

    prb = prb_ref[...]
    ecx = ((gx1 + gx2) * 0.5 - prb[0:1, :]) * prb[2:3, :]
    ecy = ((gy1 + gy2) * 0.5 - prb[1:2, :]) * prb[3:4, :]
    ew = jnp.log(jnp.maximum((gx2 - gx1) * prb[4:5, :], 1e-30)) * (1.0 / VAR1)
    eh = jnp.log(jnp.maximum((gy2 - gy1) * prb[5:6, :], 1.0e-30)) * (1.0 / VAR1)
    locr = loc_ref[0]

    def huber(d):
        ad = jnp.abs(d)
        return jnp.where(ad < 1.0, 0.5 * d * d, ad - 0.5)

    sl1 = (huber(locr[0:1, :] - ecx) + huber(locr[1:2, :] - ecy)
           + huber(locr[2:3, :] - ew) + huber(locr[3:4, :] - eh))
    loss_l = jnp.sum(jnp.where(pos, sl1, 0.0))
    npos = jnp.sum(pos.astype(jnp.float32))

    cf = conf_ref[0]
    lse = jnp.log(jnp.sum(jnp.exp(cf), axis=0, keepdims=True))
    cid = jax.lax.broadcasted_iota(jnp.int32, (CP, LBLK), 0)
    gat = jnp.sum(jnp.where(cid == conf_t, cf, 0.0), axis=0, keepdims=True)
    ce = lse - gat
    pos_ce = jnp.sum(jnp.where(pos, ce, 0.0))
    padmask = pidx[0:1, :] >= P
    lc_ref[0] = jnp.where(pos | padmask, 0.0, ce)

    part = jnp.concatenate([
        loss_l.reshape(1, 1), pos_ce.reshape(1, 1), npos.reshape(1, 1),
        jnp.zeros((1, 5), jnp.float32)], axis=1)

    @pl.when(c == 0)
    def _():
        stats_ref[0] = jnp.zeros((1, 8), jnp.float32)

    stats_ref[0] = stats_ref[0] + part


NC, NS, L = 2, 16, 16
NV = PP // L


def _sc_topk(lc_hbm, kb_hbm, out_hbm, row_v, kb_v, out_v):
    wid = lax.axis_index("s") * NC + lax.axis_index("c")
    pltpu.sync_copy(lc_hbm.at[wid], row_v)
    pltpu.sync_copy(kb_hbm.at[wid], kb_v)
    kvec = kb_v[...]

    def count_gt(midv):
        tfv = plsc.bitcast(midv, jnp.float32)

        def body(i, acc):
            v = row_v[pl.ds(i, L)]
            return acc + plsc.all_reduce_population_count(v > tfv)

        return plsc.parallel_loop(
            0, PP, L, unroll=16, carry=jnp.zeros((L,), jnp.int32))(body)

    def bs_body(_, carry):
        lo, hi = carry
        mid = lo + jnp.right_shift(hi - lo, 1)
        take_hi = count_gt(mid) < kvec
        return (jnp.where(take_hi, lo, mid + 1),
                jnp.where(take_hi, mid, hi))

    zi = jnp.zeros((L,), jnp.int32)
    lo, _ = lax.fori_loop(0, 31, bs_body, (zi, zi + 0x7F800000))
    tfv = plsc.bitcast(lo, jnp.float32)

    zf = jnp.zeros((L,), jnp.float32)

    def fbody(i, carry):
        s, cn, tm = carry
        v = row_v[pl.ds(i, L)]
        gt = v > tfv
        return (s + jnp.where(gt, v, 0.0),
                cn + gt.astype(jnp.float32),
                jnp.maximum(tm, jnp.where(v == tfv, v, 0.0)))

    s, cn, tm = plsc.parallel_loop(
        0, PP, L, unroll=8, carry=(zf, zf, zf))(fbody)
    out_v[pl.ds(0, L)] = s
    out_v[pl.ds(L, L)] = cn
    out_v[pl.ds(2 * L, L)] = tm
    pltpu.sync_copy(out_v, out_hbm.at[wid])


def _combine_d(stats_ref, tk_ref, out_ref):
    tk = tk_ref[...]
    sum_gt = jnp.sum(tk[:, 0:L], axis=1, keepdims=True)
    cnt_gt = jnp.sum(tk[:, L:2 * L], axis=1, keepdims=True)
    tval = jnp.max(tk[:, 2 * L:3 * L], axis=1, keepdims=True)
    npos = stats_ref[:, 2:3]
    kf = jnp.minimum(jnp.float32(NEGPOS_RATIO) * npos, jnp.float32(P - 1))
    topk = sum_gt + (kf - cnt_gt) * tval
    n = jnp.sum(npos)
    loss_l = jnp.sum(stats_ref[:, 0:1])
    loss_c = jnp.sum(stats_ref[:, 1:2] + topk)
    out_ref[...] = jnp.concatenate([
        (loss_l / n).reshape(1), (loss_c / n).reshape(1),
        jnp.zeros((6,), jnp.float32)])


@jax.jit
def kernel(loc_data, conf_data, priors, targets):
    f32 = jnp.float32
    boxes = targets[:, :, :4]
    labels = targets[:, :, 4]
    t8 = jnp.zeros((B, 8, OP), f32)
    t8 = t8.at[:, 0:4, :O].set(jnp.transpose(boxes, (0, 2, 1)))
    t8 = t8.at[:, 4, :O].set(labels)
    t8 = t8.at[:, 5, :O].set(1.0)
    t8b = jnp.transpose(t8, (0, 2, 1))

    cx, cy, w, h = priors[:, 0], priors[:, 1], priors[:, 2], priors[:, 3]
    pra = jnp.zeros((8, PP), f32)
    pra = pra.at[0, :P].set(cx - w * 0.5)
    pra = pra.at[1, :P].set(cy - h * 0.5)
    pra = pra.at[2, :P].set(cx + w * 0.5)
    pra = pra.at[3, :P].set(cy + h * 0.5)
    pra = pra.at[4, :P].set(w * h)
    pra = pra.at[0:4, P:].set(-10.0)
    prb = jnp.ones((8, PP), f32)
    prb = prb.at[0, :P].set(cx)
    prb = prb.at[1, :P].set(cy)
    prb = prb.at[2, :P].set(1.0 / (VAR0 * w))
    prb = prb.at[3, :P].set(1.0 / (VAR0 * h))
    prb = prb.at[4, :P].set(1.0 / w)
    prb = prb.at[5, :P].set(1.0 / h)

    loc_t = jnp.zeros((B, 8, PP), f32)
    loc_t = loc_t.at[:, 0:4, :P].set(jnp.transpose(loc_data, (0, 2, 1)))
    conf_t_in = jnp.full((B, CP, PP), -1e30, f32)
    conf_t_in = conf_t_in.at[:, :C, :P].set(jnp.transpose(conf_data, (0, 2, 1)))

    bto, bti, bpi = pl.pallas_call(
        _match_a,
        grid=(B, NCH),
        in_specs=[
            pl.BlockSpec((1, OP, 8), lambda b, c: (b, 0, 0)),
            pl.BlockSpec((8, LBLK), lambda b, c: (0, c)),
        ],
        out_specs=[
            pl.BlockSpec((1, 1, LBLK), lambda b, c: (b * NCH + c, 0, 0)),
            pl.BlockSpec((1, 1, LBLK), lambda b, c: (b * NCH + c, 0, 0)),
            pl.BlockSpec((1, OP, 1), lambda b, c: (b, 0, 0)),
        ],
        out_shape=[
            jax.ShapeDtypeStruct((B * NCH, 1, LBLK), f32),
            jax.ShapeDtypeStruct((B * NCH, 1, LBLK), jnp.int32),
            jax.ShapeDtypeStruct((B, OP, 1), jnp.int32),
        ],
        scratch_shapes=[
            pltpu.VMEM((OP, 1), f32),
            pltpu.VMEM((OP, 1), jnp.int32),
        ],
    )(t8b, pra)

    lc, stats = pl.pallas_call(
        _match_b,
        grid=(B, NCH),
        in_specs=[
            pl.BlockSpec((1, 8, OP), lambda b, c: (b, 0, 0)),
            pl.BlockSpec((1, OP, 8), lambda b, c: (b, 0, 0)),
            pl.BlockSpec((8, LBLK), lambda b, c: (0, c)),
            pl.BlockSpec((1, 8, LBLK), lambda b, c: (b, 0, c)),
            pl.BlockSpec((1, CP, LBLK), lambda b, c: (b, 0, c)),
            pl.BlockSpec((1, 1, LBLK), lambda b, c: (b * NCH + c, 0, 0)),
            pl.BlockSpec((1, 1, LBLK), lambda b, c: (b * NCH + c, 0, 0)),
            pl.BlockSpec((1, OP, 1), lambda b, c: (b, 0, 0)),
        ],
        out_specs=[
            pl.BlockSpec((1, 1, LBLK), lambda b, c: (b * NCH + c, 0, 0)),
            pl.BlockSpec((1, 1, 8), lambda b, c: (b, 0, 0)),
        ],
        out_shape=[
            jax.ShapeDtypeStruct((B * NCH, 1, LBLK), f32),
            jax.ShapeDtypeStruct((B, 1, 8), f32),
        ],
    )(t8, t8b, prb, loc_t, conf_t_in, bto, bti, bpi)

    stats2 = stats.reshape(B, 8)
    kb = jnp.minimum(jnp.float32(NEGPOS_RATIO) * stats2[:, 2:3],
                     jnp.float32(P - 1)).astype(jnp.int32)
    kb = jnp.broadcast_to(kb, (B, L))

    sc_topk = functools.partial(
        pl.kernel,
        mesh=plsc.VectorSubcoreMesh(core_axis_name="c", subcore_axis_name="s"),
        compiler_params=pltpu.CompilerParams(needs_layout_passes=False),
        out_type=jax.ShapeDtypeStruct((B, 3 * L), f32),
        scratch_types=[
            pltpu.VMEM((PP,), f32),
            pltpu.VMEM((L,), jnp.int32),
            pltpu.VMEM((3 * L,), f32),
        ],
    )(_sc_topk)
    tk = sc_topk(lc.reshape(B, PP), kb)

    out = pl.pallas_call(
        _combine_d,
        in_specs=[
            pl.BlockSpec((B, 8), lambda: (0, 0)),
            pl.BlockSpec((B, 3 * L), lambda: (0, 0)),
        ],
        out_specs=pl.BlockSpec((8,), lambda: (0,)),
        out_shape=jax.ShapeDtypeStruct((8,), f32),
    )(stats2, tk)
    return out[0:2]
